# baseline (speedup 1.0000x reference)
import functools

import jax
import jax.numpy as jnp
from jax import lax
from jax.experimental import pallas as pl
from jax.experimental.pallas import tpu as pltpu
from jax.experimental.pallas import tpu_sc as plsc

BATCH = 16384
NUM_OBJ = 10
EMBED_DIM = 32
NUM_CLASSES = 1000
FEAT = NUM_OBJ * EMBED_DIM
NUM_ROWS = BATCH * NUM_OBJ
LINE = 128
ROWS_PER_LINE = LINE // EMBED_DIM
NUM_EMBED_LINES = 1000000 * EMBED_DIM // LINE

NUM_CORES = 2
NUM_SUBCORES = 16
NW = NUM_CORES * NUM_SUBCORES
ROWS_PER_W = NUM_ROWS // NW
CHUNK = 80
BROWS = CHUNK // NUM_OBJ
NCHUNK = ROWS_PER_W // CHUNK
NBUF = 4
GROUPS = CHUNK // 16


def _gather_body(idx_hbm, off_hbm, table_hbm, out_hbm,
                 idx_v, off_v, gbufs, obufs, semg, semf):
    wid = lax.axis_index("s") * NUM_CORES + lax.axis_index("c")
    base = wid * (BATCH // NW)
    pltpu.sync_copy(idx_hbm.at[wid], idx_v)
    pltpu.sync_copy(off_hbm.at[wid], off_v)
    iota = lax.broadcasted_iota(jnp.int32, (16,), 0)

    def start_gather(j, p):
        pltpu.async_copy(table_hbm.at[idx_v.at[j]], gbufs[p], semg[p])

    def wait_gather(p):
        pltpu.make_async_copy(
            table_hbm.at[idx_v.at[0]], gbufs[p], semg[p]).wait()

    def start_flush(j, p):
        pltpu.async_copy(
            obufs[p], out_hbm.at[pl.ds(base + j * BROWS, BROWS)], semf[p])

    def wait_flush(j, p):
        pltpu.make_async_copy(
            obufs[p], out_hbm.at[pl.ds(base + j * BROWS, BROWS)],
            semf[p]).wait()

    def extract(j, p):
        gbuf, obuf = gbufs[p], obufs[p]
        for g in range(GROUPS):
            o16 = off_v[j, pl.ds(g * 16, 16)]
            for r16 in range(16):
                row = g * 16 + r16
                r8, c0 = row // NUM_OBJ, (row % NUM_OBJ) * EMBED_DIM
                ob_ = lax.gather(
                    o16, jnp.full((16, 1), r16, jnp.int32),
                    lax.GatherDimensionNumbers(
                        offset_dims=(), collapsed_slice_dims=(0,),
                        start_index_map=(0,)),
                    (1,), mode=lax.GatherScatterMode.PROMISE_IN_BOUNDS)
                rows = jnp.full((16,), row, jnp.int32)
                for h in range(EMBED_DIM // 16):
                    vals = plsc.load_gather(
                        gbuf, [rows, ob_ + h * 16 + iota])
                    obuf[r8, pl.ds(c0 + h * 16, 16)] = vals

    for p in range(NBUF - 1):
        start_gather(p, p)

    def outer(io, _):
        j0 = NBUF * io
        for p in range(NBUF):
            j = j0 + p
            @pl.when(j + NBUF - 1 < NCHUNK)
            def _():
                start_gather(j + NBUF - 1, (p + NBUF - 1) % NBUF)

            wait_gather(p)

            @pl.when(io > 0)
            def _():
                wait_flush(j - NBUF, p)

            extract(j, p)
            start_flush(j, p)
        return 0

    lax.fori_loop(0, NCHUNK // NBUF, outer, 0)
    for p in range(NBUF):
        wait_flush(NCHUNK - NBUF + p, p)


def _sc_gather(idx, off, table2d):
    mesh = plsc.VectorSubcoreMesh(core_axis_name="c", subcore_axis_name="s")
    k = functools.partial(
        pl.kernel,
        mesh=mesh,
        out_type=jax.ShapeDtypeStruct((BATCH, FEAT), jnp.float32),
        scratch_types=[
            pltpu.VMEM((NCHUNK, CHUNK), jnp.int32),
            pltpu.VMEM((NCHUNK, CHUNK), jnp.int32),
            [pltpu.VMEM((CHUNK, LINE), jnp.float32) for _ in range(NBUF)],
            [pltpu.VMEM((BROWS, FEAT), jnp.float32) for _ in range(NBUF)],
            [pltpu.SemaphoreType.DMA for _ in range(NBUF)],
            [pltpu.SemaphoreType.DMA for _ in range(NBUF)],
        ],
        compiler_params=pltpu.CompilerParams(needs_layout_passes=False),
    )(_gather_body)
    return k(idx, off, table2d)


M_BLK = 2048


def _mm_body(w_ref, a_ref, b_ref, o_ref):
    o_ref[...] = (
        lax.dot_general(
            w_ref[...], a_ref[...],
            (((1,), (1,)), ((), ())),
            preferred_element_type=jnp.float32,
        )
        + b_ref[...]
    )


def _tc_matmul_t(flat, w, bcol):
    grid = (BATCH // M_BLK,)
    return pl.pallas_call(
        _mm_body,
        grid=grid,
        in_specs=[
            pl.BlockSpec((NUM_CLASSES, FEAT), lambda i: (0, 0)),
            pl.BlockSpec((M_BLK, FEAT), lambda i: (i, 0)),
            pl.BlockSpec((NUM_CLASSES, 1), lambda i: (0, 0)),
        ],
        out_specs=pl.BlockSpec((NUM_CLASSES, M_BLK), lambda i: (0, i)),
        out_shape=jax.ShapeDtypeStruct((NUM_CLASSES, BATCH), jnp.float32),
        compiler_params=pltpu.CompilerParams(
            dimension_semantics=("arbitrary",),
        ),
    )(w, flat, bcol)


def kernel(x, table, W, b):
    flat_idx = x.reshape(NW, NCHUNK, CHUNK)
    line_idx = flat_idx // ROWS_PER_LINE
    lane_off = (flat_idx % ROWS_PER_LINE) * EMBED_DIM
    table2d = jnp.concatenate(
        [table[k::ROWS_PER_LINE] for k in range(ROWS_PER_LINE)], axis=1)
    flat = _sc_gather(line_idx, lane_off, table2d)
    out_t = _tc_matmul_t(flat, W, b.reshape(NUM_CLASSES, 1))
    return out_t.T


# speedup vs baseline: 7.5023x; 7.5023x over previous
import functools

import jax
import jax.numpy as jnp
from jax import lax
from jax.experimental import pallas as pl
from jax.experimental.pallas import tpu as pltpu
from jax.experimental.pallas import tpu_sc as plsc

BATCH = 16384
NUM_OBJ = 10
EMBED_DIM = 32
NUM_CLASSES = 1000
FEAT = NUM_OBJ * EMBED_DIM
NUM_ROWS = BATCH * NUM_OBJ
LINE = 128
ROWS_PER_LINE = LINE // EMBED_DIM
NUM_EMBED_LINES = 1000000 * EMBED_DIM // LINE

NUM_CORES = 2
NUM_SUBCORES = 16
NW = NUM_CORES * NUM_SUBCORES
ROWS_PER_W = NUM_ROWS // NW
CHUNK = 80
BROWS = CHUNK // NUM_OBJ
NCHUNK = ROWS_PER_W // CHUNK
NBUF = 4
GROUPS = CHUNK // 16


def _gather_body(idx_hbm, off_hbm, table_hbm, out_hbm,
                 idx_v, off_v, gbufs, obufs, semg, semf):
    wid = lax.axis_index("s") * NUM_CORES + lax.axis_index("c")
    base = wid * (BATCH // NW)
    pltpu.sync_copy(idx_hbm.at[wid], idx_v)
    pltpu.sync_copy(off_hbm.at[wid], off_v)
    iota = lax.broadcasted_iota(jnp.int32, (16,), 0)

    def start_gather(j, p):
        pltpu.async_copy(table_hbm.at[idx_v.at[j]], gbufs[p], semg[p])

    def wait_gather(p):
        pltpu.make_async_copy(
            table_hbm.at[idx_v.at[0]], gbufs[p], semg[p]).wait()

    def start_flush(j, p):
        pltpu.async_copy(
            obufs[p], out_hbm.at[pl.ds(base + j * BROWS, BROWS)], semf[p])

    def wait_flush(j, p):
        pltpu.make_async_copy(
            obufs[p], out_hbm.at[pl.ds(base + j * BROWS, BROWS)],
            semf[p]).wait()

    def extract(j, p):
        gbuf, obuf = gbufs[p], obufs[p]
        for g in range(GROUPS):
            o16 = off_v[j, pl.ds(g * 16, 16)]
            for r16 in range(16):
                row = g * 16 + r16
                r8, c0 = row // NUM_OBJ, (row % NUM_OBJ) * EMBED_DIM
                ob_ = lax.gather(
                    o16, jnp.full((16, 1), r16, jnp.int32),
                    lax.GatherDimensionNumbers(
                        offset_dims=(), collapsed_slice_dims=(0,),
                        start_index_map=(0,)),
                    (1,), mode=lax.GatherScatterMode.PROMISE_IN_BOUNDS)
                rows = jnp.full((16,), row, jnp.int32)
                for h in range(EMBED_DIM // 16):
                    vals = plsc.load_gather(
                        gbuf, [rows, ob_ + h * 16 + iota])
                    obuf[r8, pl.ds(c0 + h * 16, 16)] = vals

    for p in range(NBUF - 1):
        start_gather(p, p)

    def outer(io, _):
        j0 = NBUF * io
        for p in range(NBUF):
            j = j0 + p
            @pl.when(j + NBUF - 1 < NCHUNK)
            def _():
                start_gather(j + NBUF - 1, (p + NBUF - 1) % NBUF)

            wait_gather(p)

            @pl.when(io > 0)
            def _():
                wait_flush(j - NBUF, p)

            extract(j, p)
            start_flush(j, p)
        return 0

    lax.fori_loop(0, NCHUNK // NBUF, outer, 0)
    for p in range(NBUF):
        wait_flush(NCHUNK - NBUF + p, p)


def _sc_gather(idx, off, table2d):
    mesh = plsc.VectorSubcoreMesh(core_axis_name="c", subcore_axis_name="s")
    k = functools.partial(
        pl.kernel,
        mesh=mesh,
        out_type=jax.ShapeDtypeStruct((BATCH, FEAT), jnp.float32),
        scratch_types=[
            pltpu.VMEM((NCHUNK, CHUNK), jnp.int32),
            pltpu.VMEM((NCHUNK, CHUNK), jnp.int32),
            [pltpu.VMEM((CHUNK, LINE), jnp.float32) for _ in range(NBUF)],
            [pltpu.VMEM((BROWS, FEAT), jnp.float32) for _ in range(NBUF)],
            [pltpu.SemaphoreType.DMA for _ in range(NBUF)],
            [pltpu.SemaphoreType.DMA for _ in range(NBUF)],
        ],
        compiler_params=pltpu.CompilerParams(needs_layout_passes=False),
    )(_gather_body)
    return k(idx, off, table2d)


M_BLK = 2048


def _mm_body(w_ref, a_ref, b_ref, o_ref):
    o_ref[...] = (
        lax.dot_general(
            w_ref[...], a_ref[...],
            (((1,), (1,)), ((), ())),
            preferred_element_type=jnp.float32,
        )
        + b_ref[...]
    )


def _tc_matmul_t(flat, w, bcol):
    grid = (BATCH // M_BLK,)
    return pl.pallas_call(
        _mm_body,
        grid=grid,
        in_specs=[
            pl.BlockSpec((NUM_CLASSES, FEAT), lambda i: (0, 0)),
            pl.BlockSpec((M_BLK, FEAT), lambda i: (i, 0)),
            pl.BlockSpec((NUM_CLASSES, 1), lambda i: (0, 0)),
        ],
        out_specs=pl.BlockSpec((NUM_CLASSES, M_BLK), lambda i: (0, i)),
        out_shape=jax.ShapeDtypeStruct((NUM_CLASSES, BATCH), jnp.float32),
        compiler_params=pltpu.CompilerParams(
            dimension_semantics=("arbitrary",),
        ),
    )(w, flat, bcol)


def kernel(x, table, W, b):
    flat_idx = x.reshape(NW, NCHUNK, CHUNK)
    line_idx = flat_idx // ROWS_PER_LINE
    lane_off = (flat_idx % ROWS_PER_LINE) * EMBED_DIM
    table2d = table.reshape(NUM_EMBED_LINES, LINE)
    flat = _sc_gather(line_idx, lane_off, table2d)
    out_t = _tc_matmul_t(flat, W, b.reshape(NUM_CLASSES, 1))
    return out_t.T
